# baseline (speedup 1.0000x reference)
import functools

import jax
import jax.numpy as jnp
from jax import lax
from jax.experimental import pallas as pl
from jax.experimental.pallas import tpu as pltpu
from jax.experimental.pallas import tpu_sc as plsc

NC, NS = 2, 16
NW = NC * NS
CHUNK = 128
NB = 4
L = 16


@functools.partial(jax.jit, static_argnums=(2, 3, 4))
def _sc_gather_t(tok2d, table, s0, s1, D):
    n_chunks = tok2d.shape[0]
    cpw = n_chunks // NW
    bpj = s0 // CHUNK
    assert cpw % NB == 0 and cpw // NB >= 3
    n_grps = cpw // NB
    mesh = plsc.VectorSubcoreMesh(core_axis_name="c", subcore_axis_name="s")

    @functools.partial(
        pl.kernel,
        out_type=jax.ShapeDtypeStruct((s1, D, s0), jnp.float32),
        mesh=mesh,
        scratch_types=[
            pltpu.VMEM((cpw, CHUNK), jnp.int32),
            pltpu.VMEM((NB, CHUNK, D), jnp.float32),
            pltpu.VMEM((NB, D, CHUNK), jnp.float32),
            pltpu.SemaphoreType.DMA,
            pltpu.SemaphoreType.DMA,
        ],
        compiler_params=pltpu.CompilerParams(
            use_tc_tiling_on_sc=False, needs_layout_passes=False),
    )
    def k(tok_hbm, table_hbm, out_hbm, idx_v, bufs, bufTs, sem_g, sem_w):
        wid = lax.axis_index("s") * NC + lax.axis_index("c")
        c0 = wid * cpw
        pltpu.sync_copy(tok_hbm.at[pl.ds(c0, cpw)], idx_v)

        i16 = lax.iota(jnp.int32, 16)
        rows = [i16 + L * kk for kk in range(CHUNK // L)]

        def start_gather(t, b):
            pltpu.async_copy(table_hbm.at[idx_v.at[t]], bufs.at[b], sem_g)

        def wait_gather(b):
            pltpu.make_async_copy(
                table_hbm.at[idx_v.at[0]], bufs.at[b], sem_g).wait()

        def start_write(t, b):
            cid = c0 + t
            j = cid // bpj
            b0 = (cid % bpj) * CHUNK
            pltpu.async_copy(
                bufTs.at[b], out_hbm.at[j, :, pl.ds(b0, CHUNK)], sem_w)

        def wait_write(b):
            pltpu.make_async_copy(
                bufTs.at[b], out_hbm.at[0, :, pl.ds(0, CHUNK)], sem_w).wait()

        def transpose(b):
            src = bufs.at[b]

            def dbody(d, carry):
                col = jnp.zeros((16,), jnp.int32) + d
                for kk in range(CHUNK // L):
                    v = plsc.load_gather(src, [rows[kk], col])
                    bufTs[b, d, pl.ds(kk * L, L)] = v
                return carry

            lax.fori_loop(0, D, dbody, 0, unroll=False)

        for b in range(NB):
            start_gather(b, b)
        for b in range(NB):
            wait_gather(b)
            transpose(b)
            start_write(b, b)
            start_gather(NB + b, b)

        def group(g, carry):
            t0 = g * NB
            for b in range(NB):
                wait_gather(b)
                wait_write(b)
                transpose(b)
                start_write(t0 + b, b)
                start_gather(t0 + NB + b, b)
            return carry

        lax.fori_loop(1, n_grps - 1, group, 0, unroll=False)

        t0 = (n_grps - 1) * NB
        for b in range(NB):
            wait_gather(b)
            wait_write(b)
            transpose(b)
            start_write(t0 + b, b)
        for b in range(NB):
            wait_write(b)

    return k(tok2d, table)


def kernel(token_ids, embedding_lookup):
    s0, s1 = token_ids.shape
    D = embedding_lookup.shape[1]
    tok2d = token_ids.T.reshape((s0 * s1) // CHUNK, CHUNK)
    outP = _sc_gather_t(tok2d, embedding_lookup, s0, s1, D)
    return outP.transpose(2, 0, 1)
